# baseline (speedup 1.0000x reference)
import functools

import jax
import jax.numpy as jnp
from jax import lax
from jax.experimental import pallas as pl
from jax.experimental.pallas import tpu as pltpu
from jax.experimental.pallas import tpu_sc as plsc

_TOPK = 2
_E = 16
_CHUNK = 128


def _probs_body(x_ref, w_ref, b_ref, p_ref):
    bt = x_ref.shape[0]
    logits = lax.dot_general(
        x_ref[...], w_ref[...], (((1,), (1,)), ((), ())),
        preferred_element_type=jnp.float32) + b_ref[...]
    probs = 1.0 / (1.0 + jnp.exp(-logits))
    p_ref[...] = jnp.concatenate(
        [probs, jnp.zeros((bt, 128 - _E), jnp.float32)], axis=1)


def _make_sc_router(n):
    info = plsc.get_sparse_core_info()
    nw = info.num_cores * info.num_subcores
    rows = n // nw
    c = _CHUNK
    n_chunks = rows // c
    mesh = plsc.VectorSubcoreMesh(core_axis_name="c", subcore_axis_name="s")

    @functools.partial(
        pl.kernel,
        out_type=(
            jax.ShapeDtypeStruct((n, _TOPK), jnp.float32),
            jax.ShapeDtypeStruct((n, _TOPK), jnp.int32),
            jax.ShapeDtypeStruct((n, _E), jnp.float32),
        ),
        mesh=mesh,
        scratch_types=[
            pltpu.VMEM((rows * 128,), jnp.float32),
            pltpu.VMEM((c, _TOPK), jnp.float32),
            pltpu.VMEM((c, _TOPK), jnp.int32),
            pltpu.VMEM((c, _E), jnp.float32),
        ],
        compiler_params=pltpu.CompilerParams(needs_layout_passes=False),
    )
    def sc_router(p_hbm, topw_hbm, topi_hbm, rw_hbm, p_v, tw_v, ti_v, rw_v):
        wid = lax.axis_index("s") * info.num_cores + lax.axis_index("c")
        base = wid * rows
        pltpu.sync_copy(p_hbm.at[pl.ds(base * 128, rows * 128)], p_v)
        lane = lax.iota(jnp.int32, 16)
        mask2 = lane < _TOPK
        zero = jnp.zeros((_E,), jnp.float32)

        for chunk in range(n_chunks):
            def body(rl, _chunk=chunk):
                r = _chunk * c + rl
                row = p_v[pl.ds(r * 128, _E)]
                s, v = plsc.sort_key_val(row, lane, descending=True)
                rvec = jnp.full((16,), rl, jnp.int32)
                plsc.store_scatter(rw_v, [rvec, v], jnp.where(mask2, s, 0.0))
                plsc.store_scatter(tw_v, [rvec, lane], s, mask=mask2)
                plsc.store_scatter(ti_v, [rvec, lane], v, mask=mask2)

            plsc.parallel_loop(0, c, 1, unroll=4)(body)
            off = base + chunk * c
            pltpu.sync_copy(tw_v, topw_hbm.at[pl.ds(off, c)])
            pltpu.sync_copy(ti_v, topi_hbm.at[pl.ds(off, c)])
            pltpu.sync_copy(rw_v, rw_hbm.at[pl.ds(off, c)])

    return sc_router


def kernel(x, W, b):
    batch, seq, d = x.shape
    n = batch * seq
    xf = x.reshape(n, d)
    bt = 2048
    chunk_sizes = (8192, 4096, 4096)
    routers = {nc: _make_sc_router(nc) for nc in set(chunk_sizes)}
    parts = []
    tok0 = 0
    for nc in chunk_sizes:
        off = tok0 // bt
        probs = pl.pallas_call(
            _probs_body,
            grid=(nc // bt,),
            in_specs=[
                pl.BlockSpec((bt, d), lambda i, off=off: (off + i, 0)),
                pl.BlockSpec((_E, d), lambda i: (0, 0)),
                pl.BlockSpec((1, _E), lambda i: (0, 0)),
            ],
            out_specs=pl.BlockSpec((bt, 128), lambda i: (i, 0)),
            out_shape=jax.ShapeDtypeStruct((nc, 128), jnp.float32),
        )(xf, W, b.reshape(1, _E))
        parts.append(routers[nc](probs.reshape(-1)))
        tok0 += nc
    return tuple(
        jnp.concatenate([p[j] for p in parts], axis=0) for j in range(3))


# speedup vs baseline: 1.0112x; 1.0112x over previous
import functools

import jax
import jax.numpy as jnp
from jax import lax
from jax.experimental import pallas as pl
from jax.experimental.pallas import tpu as pltpu
from jax.experimental.pallas import tpu_sc as plsc

_TOPK = 2
_E = 16
_CHUNK = 128


def _probs_body(x_ref, w_ref, b_ref, p_ref):
    bt = x_ref.shape[0]
    logits = lax.dot_general(
        x_ref[...], w_ref[...], (((1,), (1,)), ((), ())),
        preferred_element_type=jnp.float32) + b_ref[...]
    probs = 1.0 / (1.0 + jnp.exp(-logits))
    p_ref[...] = jnp.concatenate(
        [probs, jnp.zeros((bt, 128 - _E), jnp.float32)], axis=1)


def _make_sc_router(n):
    info = plsc.get_sparse_core_info()
    nw = info.num_cores * info.num_subcores
    rows = n // nw
    c = _CHUNK
    n_chunks = rows // c
    mesh = plsc.VectorSubcoreMesh(core_axis_name="c", subcore_axis_name="s")

    @functools.partial(
        pl.kernel,
        out_type=(
            jax.ShapeDtypeStruct((n, _TOPK), jnp.float32),
            jax.ShapeDtypeStruct((n, _TOPK), jnp.int32),
            jax.ShapeDtypeStruct((n, _E), jnp.float32),
        ),
        mesh=mesh,
        scratch_types=[
            pltpu.VMEM((rows * 128,), jnp.float32),
            pltpu.VMEM((c, _TOPK), jnp.float32),
            pltpu.VMEM((c, _TOPK), jnp.int32),
            pltpu.VMEM((c, _E), jnp.float32),
        ],
        compiler_params=pltpu.CompilerParams(needs_layout_passes=False),
    )
    def sc_router(p_hbm, topw_hbm, topi_hbm, rw_hbm, p_v, tw_v, ti_v, rw_v):
        wid = lax.axis_index("s") * info.num_cores + lax.axis_index("c")
        base = wid * rows
        pltpu.sync_copy(p_hbm.at[pl.ds(base * 128, rows * 128)], p_v)
        lane = lax.iota(jnp.int32, 16)
        mask2 = lane < _TOPK
        zero = jnp.zeros((_E,), jnp.float32)

        for chunk in range(n_chunks):
            def body(rl, _chunk=chunk):
                r = _chunk * c + rl
                row = p_v[pl.ds(r * 128, _E)]
                s, v = plsc.sort_key_val(row, lane, descending=True)
                rvec = jnp.full((16,), rl, jnp.int32)
                plsc.store_scatter(rw_v, [rvec, v], jnp.where(mask2, s, 0.0))
                plsc.store_scatter(tw_v, [rvec, lane], s, mask=mask2)
                plsc.store_scatter(ti_v, [rvec, lane], v, mask=mask2)

            plsc.parallel_loop(0, c, 1, unroll=4)(body)
            off = base + chunk * c
            pltpu.sync_copy(tw_v, topw_hbm.at[pl.ds(off, c)])
            pltpu.sync_copy(ti_v, topi_hbm.at[pl.ds(off, c)])
            pltpu.sync_copy(rw_v, rw_hbm.at[pl.ds(off, c)])

    return sc_router


def kernel(x, W, b):
    batch, seq, d = x.shape
    n = batch * seq
    xf = x.reshape(n, d)
    bt = 2048
    chunk_sizes = (12288, 4096)
    routers = {nc: _make_sc_router(nc) for nc in set(chunk_sizes)}
    parts = []
    tok0 = 0
    for nc in chunk_sizes:
        off = tok0 // bt
        probs = pl.pallas_call(
            _probs_body,
            grid=(nc // bt,),
            in_specs=[
                pl.BlockSpec((bt, d), lambda i, off=off: (off + i, 0)),
                pl.BlockSpec((_E, d), lambda i: (0, 0)),
                pl.BlockSpec((1, _E), lambda i: (0, 0)),
            ],
            out_specs=pl.BlockSpec((bt, 128), lambda i: (i, 0)),
            out_shape=jax.ShapeDtypeStruct((nc, 128), jnp.float32),
        )(xf, W, b.reshape(1, _E))
        parts.append(routers[nc](probs.reshape(-1)))
        tok0 += nc
    return tuple(
        jnp.concatenate([p[j] for p in parts], axis=0) for j in range(3))


# speedup vs baseline: 1.0559x; 1.0442x over previous
import functools

import jax
import jax.numpy as jnp
from jax import lax
from jax.experimental import pallas as pl
from jax.experimental.pallas import tpu as pltpu
from jax.experimental.pallas import tpu_sc as plsc

_TOPK = 2
_E = 16
_CHUNK = 128


def _probs_body(x_ref, w_ref, b_ref, p_ref):
    bt = x_ref.shape[0]
    logits = lax.dot_general(
        x_ref[...], w_ref[...], (((1,), (1,)), ((), ())),
        preferred_element_type=jnp.float32) + b_ref[...]
    probs = 1.0 / (1.0 + jnp.exp(-logits))
    p_ref[...] = jnp.concatenate(
        [probs, jnp.zeros((bt, 128 - _E), jnp.float32)], axis=1)


def _make_sc_router(n):
    info = plsc.get_sparse_core_info()
    nw = info.num_cores * info.num_subcores
    rows = n // nw
    c = _CHUNK
    n_chunks = rows // c
    mesh = plsc.VectorSubcoreMesh(core_axis_name="c", subcore_axis_name="s")

    @functools.partial(
        pl.kernel,
        out_type=(
            jax.ShapeDtypeStruct((n, _TOPK), jnp.float32),
            jax.ShapeDtypeStruct((n, _TOPK), jnp.int32),
            jax.ShapeDtypeStruct((n, _E), jnp.float32),
        ),
        mesh=mesh,
        scratch_types=[
            pltpu.VMEM((rows * 128,), jnp.float32),
            pltpu.VMEM((c, _TOPK), jnp.float32),
            pltpu.VMEM((c, _TOPK), jnp.int32),
            pltpu.VMEM((c, _E), jnp.float32),
        ],
        compiler_params=pltpu.CompilerParams(needs_layout_passes=False),
    )
    def sc_router(p_hbm, topw_hbm, topi_hbm, rw_hbm, p_v, tw_v, ti_v, rw_v):
        wid = lax.axis_index("s") * info.num_cores + lax.axis_index("c")
        base = wid * rows
        pltpu.sync_copy(p_hbm.at[pl.ds(base * 128, rows * 128)], p_v)
        lane = lax.iota(jnp.int32, 16)
        mask2 = lane < _TOPK
        zero = jnp.zeros((_E,), jnp.float32)

        for chunk in range(n_chunks):
            def body(rl, _chunk=chunk):
                r = _chunk * c + rl
                row = p_v[pl.ds(r * 128, _E)]
                s, v = plsc.sort_key_val(row, lane, descending=True)
                rvec = jnp.full((16,), rl, jnp.int32)
                plsc.store_scatter(rw_v, [rvec, v], jnp.where(mask2, s, 0.0))
                plsc.store_scatter(tw_v, [rvec, lane], s, mask=mask2)
                plsc.store_scatter(ti_v, [rvec, lane], v, mask=mask2)

            plsc.parallel_loop(0, c, 1, unroll=8)(body)
            off = base + chunk * c
            pltpu.sync_copy(tw_v, topw_hbm.at[pl.ds(off, c)])
            pltpu.sync_copy(ti_v, topi_hbm.at[pl.ds(off, c)])
            pltpu.sync_copy(rw_v, rw_hbm.at[pl.ds(off, c)])

    return sc_router


def kernel(x, W, b):
    batch, seq, d = x.shape
    n = batch * seq
    xf = x.reshape(n, d)
    bt = 2048
    chunk_sizes = (8192, 8192)
    routers = {nc: _make_sc_router(nc) for nc in set(chunk_sizes)}
    parts = []
    tok0 = 0
    for nc in chunk_sizes:
        off = tok0 // bt
        probs = pl.pallas_call(
            _probs_body,
            grid=(nc // bt,),
            in_specs=[
                pl.BlockSpec((bt, d), lambda i, off=off: (off + i, 0)),
                pl.BlockSpec((_E, d), lambda i: (0, 0)),
                pl.BlockSpec((1, _E), lambda i: (0, 0)),
            ],
            out_specs=pl.BlockSpec((bt, 128), lambda i: (i, 0)),
            out_shape=jax.ShapeDtypeStruct((nc, 128), jnp.float32),
        )(xf, W, b.reshape(1, _E))
        parts.append(routers[nc](probs.reshape(-1)))
        tok0 += nc
    return tuple(
        jnp.concatenate([p[j] for p in parts], axis=0) for j in range(3))
